# baseline (speedup 1.0000x reference)
import functools

import jax
import jax.numpy as jnp
from jax import lax
from jax.experimental import pallas as pl
from jax.experimental.pallas import tpu as pltpu

_EPS = 1e-12
_K = 20


def _stats_loss_kernel(xa_ref, xb_ref, t_ref, out_ref, sa_ref, sb_ref,
                       na_ref, nb_ref, counts_ref, *, num_classes, n_steps):
    K = num_classes
    q = pl.program_id(0)

    @pl.when(q == 0)
    def _():
        sa_ref[...] = jnp.zeros_like(sa_ref)
        sb_ref[...] = jnp.zeros_like(sb_ref)
        na_ref[...] = jnp.zeros_like(na_ref)
        nb_ref[...] = jnp.zeros_like(nb_ref)
        counts_ref[...] = jnp.zeros_like(counts_ref)

    t = t_ref[...]
    Ch, R, W = xa_ref.shape
    T = R * W

    xa = xa_ref[...].astype(jnp.float32).reshape(Ch, T)
    xb = xb_ref[...].astype(jnp.float32).reshape(Ch, T)
    tflat = t.reshape(1, T)

    ones_row = jnp.ones((1, Ch), jnp.float32)
    cdims = (((1,), (0,)), ((), ()))
    norm2 = (lax.dot_general(ones_row, xa * xa, cdims,
                             preferred_element_type=jnp.float32)
             + lax.dot_general(ones_row, xb * xb, cdims,
                               preferred_element_type=jnp.float32))
    invnorm = lax.rsqrt(norm2 + _EPS)

    class_ids = lax.broadcasted_iota(jnp.int32, (K, T), 0)
    mask = class_ids == tflat
    onehot = jnp.where(mask, 1.0, 0.0).astype(jnp.float32)
    scaled = jnp.where(mask, jnp.broadcast_to(invnorm, (K, T)),
                       0.0).astype(jnp.float32)

    dims = (((1,), (1,)), ((), ()))
    sa_ref[...] += lax.dot_general(onehot, xa, dims,
                                   preferred_element_type=jnp.float32)
    sb_ref[...] += lax.dot_general(onehot, xb, dims,
                                   preferred_element_type=jnp.float32)
    na_ref[...] += lax.dot_general(scaled, xa, dims,
                                   preferred_element_type=jnp.float32)
    nb_ref[...] += lax.dot_general(scaled, xb, dims,
                                   preferred_element_type=jnp.float32)
    counts_ref[...] += jnp.sum(onehot, axis=1, keepdims=True)

    @pl.when(q == n_steps - 1)
    def _():
        sums = jnp.concatenate([sa_ref[...], sb_ref[...]], axis=1)
        nsums = jnp.concatenate([na_ref[...], nb_ref[...]], axis=1)
        counts = counts_ref[...]

        valid = counts > 0.0
        sum_pixel = jnp.maximum(counts, 1.0)
        centers = sums / sum_pixel

        cn2 = jnp.sum(centers * centers, axis=1, keepdims=True) + _EPS
        norms = jnp.sqrt(cn2)
        rn = 1.0 / norms

        dot_nc = jnp.sum(nsums * centers, axis=1, keepdims=True)
        sim_per = 1.0 - dot_nc / (norms * sum_pixel)
        sim_loss = jnp.sum(jnp.where(valid, sim_per, 0.0), keepdims=True)

        gram = lax.dot_general(centers, centers, (((1,), (1,)), ((), ())),
                               preferred_element_type=jnp.float32)
        ids_r = lax.broadcasted_iota(jnp.int32, (K, K), 0)
        ids_c = lax.broadcasted_iota(jnp.int32, (K, K), 1)
        offdiag = jnp.where(ids_r == ids_c, 0.0, jnp.maximum(gram, 0.0))
        colsum = lax.dot_general(offdiag, rn, (((1,), (0,)), ((), ())),
                                 preferred_element_type=jnp.float32)
        diag_cos = (cn2 - _EPS) / cn2
        per_row = (colsum * rn + (1.0 - diag_cos)) / K
        diff_loss = jnp.sum(jnp.where(valid, per_row, 0.0), keepdims=True)

        out_ref[...] = sim_loss + diff_loss


def _embedding_loss(inputs_nchw, targets, num_classes, *, tile_rows=64):
    N, C, H, W = inputs_nchw.shape
    K = num_classes
    Ch = C // 2
    t = targets.astype(jnp.int32)

    R = tile_rows
    while R > 8 and H % R != 0:
        R //= 2
    if H % R != 0:
        R = H
    n_tiles = H // R
    n_steps = N * n_tiles

    loss = pl.pallas_call(
        functools.partial(_stats_loss_kernel, num_classes=K, n_steps=n_steps),
        out_shape=jax.ShapeDtypeStruct((1, 1), jnp.float32),
        grid_spec=pltpu.PrefetchScalarGridSpec(
            num_scalar_prefetch=0, grid=(n_steps,),
            in_specs=[pl.BlockSpec((None, Ch, R, W),
                                   lambda q: (q // n_tiles, 0,
                                              q % n_tiles, 0)),
                      pl.BlockSpec((None, Ch, R, W),
                                   lambda q: (q // n_tiles, 1,
                                              q % n_tiles, 0)),
                      pl.BlockSpec((None, R, W),
                                   lambda q: (q // n_tiles, q % n_tiles, 0))],
            out_specs=pl.BlockSpec((1, 1), lambda q: (0, 0)),
            scratch_shapes=[pltpu.VMEM((K, Ch), jnp.float32),
                            pltpu.VMEM((K, Ch), jnp.float32),
                            pltpu.VMEM((K, Ch), jnp.float32),
                            pltpu.VMEM((K, Ch), jnp.float32),
                            pltpu.VMEM((K, 1), jnp.float32)]),
        compiler_params=pltpu.CompilerParams(
            dimension_semantics=("arbitrary",)),
    )(inputs_nchw, inputs_nchw, t)

    return loss.reshape(1)


def kernel(inputs_nchw, targets):
    return _embedding_loss(inputs_nchw, targets, _K)


# speedup vs baseline: 1.0149x; 1.0149x over previous
import functools

import jax
import jax.numpy as jnp
from jax import lax
from jax.experimental import pallas as pl
from jax.experimental.pallas import tpu as pltpu

_EPS = 1e-12
_K = 20


def _stats_loss_kernel(xa_ref, xb_ref, t_ref, out_ref, sa_ref, sb_ref,
                       na_ref, nb_ref, counts_ref, *, num_classes, n_steps):
    K = num_classes
    q = pl.program_id(0)

    @pl.when(q == 0)
    def _():
        sa_ref[...] = jnp.zeros_like(sa_ref)
        sb_ref[...] = jnp.zeros_like(sb_ref)
        na_ref[...] = jnp.zeros_like(na_ref)
        nb_ref[...] = jnp.zeros_like(nb_ref)
        counts_ref[...] = jnp.zeros_like(counts_ref)

    t = t_ref[...]
    Ch, R, W = xa_ref.shape
    T = R * W

    xa = xa_ref[...].astype(jnp.float32).reshape(Ch, T)
    xb = xb_ref[...].astype(jnp.float32).reshape(Ch, T)
    tflat = t.reshape(1, T)

    ones_row = jnp.ones((1, Ch), jnp.float32)
    cdims = (((1,), (0,)), ((), ()))
    norm2 = (lax.dot_general(ones_row, xa * xa, cdims,
                             preferred_element_type=jnp.float32)
             + lax.dot_general(ones_row, xb * xb, cdims,
                               preferred_element_type=jnp.float32))
    invnorm = lax.rsqrt(norm2 + _EPS)

    class_ids = lax.broadcasted_iota(jnp.int32, (K, T), 0)
    mask = class_ids == tflat
    onehot = jnp.where(mask, 1.0, 0.0).astype(jnp.float32)
    scaled = jnp.where(mask, jnp.broadcast_to(invnorm, (K, T)),
                       0.0).astype(jnp.float32)

    dims = (((1,), (1,)), ((), ()))
    sa_ref[...] += lax.dot_general(onehot, xa, dims,
                                   preferred_element_type=jnp.float32)
    sb_ref[...] += lax.dot_general(onehot, xb, dims,
                                   preferred_element_type=jnp.float32)
    na_ref[...] += lax.dot_general(scaled, xa, dims,
                                   preferred_element_type=jnp.float32)
    nb_ref[...] += lax.dot_general(scaled, xb, dims,
                                   preferred_element_type=jnp.float32)
    counts_ref[...] += jnp.sum(onehot, axis=1, keepdims=True)

    @pl.when(q == n_steps - 1)
    def _():
        sums = jnp.concatenate([sa_ref[...], sb_ref[...]], axis=1)
        nsums = jnp.concatenate([na_ref[...], nb_ref[...]], axis=1)
        counts = counts_ref[...]

        valid = counts > 0.0
        sum_pixel = jnp.maximum(counts, 1.0)
        centers = sums / sum_pixel

        cn2 = jnp.sum(centers * centers, axis=1, keepdims=True) + _EPS
        norms = jnp.sqrt(cn2)
        rn = 1.0 / norms

        dot_nc = jnp.sum(nsums * centers, axis=1, keepdims=True)
        sim_per = 1.0 - dot_nc / (norms * sum_pixel)
        sim_loss = jnp.sum(jnp.where(valid, sim_per, 0.0), keepdims=True)

        gram = lax.dot_general(centers, centers, (((1,), (1,)), ((), ())),
                               preferred_element_type=jnp.float32)
        ids_r = lax.broadcasted_iota(jnp.int32, (K, K), 0)
        ids_c = lax.broadcasted_iota(jnp.int32, (K, K), 1)
        offdiag = jnp.where(ids_r == ids_c, 0.0, jnp.maximum(gram, 0.0))
        colsum = lax.dot_general(offdiag, rn, (((1,), (0,)), ((), ())),
                                 preferred_element_type=jnp.float32)
        diag_cos = (cn2 - _EPS) / cn2
        per_row = (colsum * rn + (1.0 - diag_cos)) / K
        diff_loss = jnp.sum(jnp.where(valid, per_row, 0.0), keepdims=True)

        out_ref[...] = sim_loss + diff_loss


def _embedding_loss(inputs_nchw, targets, num_classes, *, tile_rows=128):
    N, C, H, W = inputs_nchw.shape
    K = num_classes
    Ch = C // 2
    t = targets.astype(jnp.int32)

    R = tile_rows
    while R > 8 and H % R != 0:
        R //= 2
    if H % R != 0:
        R = H
    n_tiles = H // R
    n_steps = N * n_tiles

    loss = pl.pallas_call(
        functools.partial(_stats_loss_kernel, num_classes=K, n_steps=n_steps),
        out_shape=jax.ShapeDtypeStruct((1, 1), jnp.float32),
        grid_spec=pltpu.PrefetchScalarGridSpec(
            num_scalar_prefetch=0, grid=(n_steps,),
            in_specs=[pl.BlockSpec((None, Ch, R, W),
                                   lambda q: (q // n_tiles, 0,
                                              q % n_tiles, 0)),
                      pl.BlockSpec((None, Ch, R, W),
                                   lambda q: (q // n_tiles, 1,
                                              q % n_tiles, 0)),
                      pl.BlockSpec((None, R, W),
                                   lambda q: (q // n_tiles, q % n_tiles, 0))],
            out_specs=pl.BlockSpec((1, 1), lambda q: (0, 0)),
            scratch_shapes=[pltpu.VMEM((K, Ch), jnp.float32),
                            pltpu.VMEM((K, Ch), jnp.float32),
                            pltpu.VMEM((K, Ch), jnp.float32),
                            pltpu.VMEM((K, Ch), jnp.float32),
                            pltpu.VMEM((K, 1), jnp.float32)]),
        compiler_params=pltpu.CompilerParams(
            dimension_semantics=("arbitrary",)),
    )(inputs_nchw, inputs_nchw, t)

    return loss.reshape(1)


def kernel(inputs_nchw, targets):
    return _embedding_loss(inputs_nchw, targets, _K)
